# baseline (speedup 1.0000x reference)
import functools

import jax
import jax.numpy as jnp
import numpy as np
from jax import lax
from jax.experimental import pallas as pl
from jax.experimental.pallas import tpu as pltpu
from jax.experimental.pallas import tpu_sc as plsc

N_HEADS = 16
N_KV = 8
HD = 128
ROPE_BASE = 500000.0

BT = 128
BN = 256
BQ = 256
BKV = 256

NW = 32
_L = 16


def _wid():
    return lax.axis_index("c") * 16 + lax.axis_index("s")


def _route_compute(t, tp, mask_hbm, pos_hbm, maskv, posv, rankv, rankl,
                   idxv, slotv, posgv, blkv):
    pltpu.sync_copy(mask_hbm, maskv)
    pltpu.sync_copy(pos_hbm, posv)
    nch = t // _L

    def p1(c, carry):
        cl, cv = carry
        m = maskv[pl.ds(c * _L, _L)]
        lm = 1 - m
        cum_v = plsc.cumsum(m)
        cum_l = plsc.cumsum(lm)
        rankv[pl.ds(c * _L, _L)] = cv + cum_v - m
        rankl[pl.ds(c * _L, _L)] = cl + cum_l - lm
        return cl + jnp.sum(lm), cv + jnp.sum(m)

    nl, nv = lax.fori_loop(0, nch, p1, (jnp.int32(0), jnp.int32(0)))
    nl_pad = ((nl + BT - 1) // BT) * BT

    def pfill(c, _):
        idxv[pl.ds(c * _L, _L)] = jnp.zeros((_L,), jnp.int32)
        return 0

    lax.fori_loop(0, tp // _L, pfill, 0)

    def p2(c, _):
        m = maskv[pl.ds(c * _L, _L)]
        s = jnp.where(m > 0, nl_pad + rankv[pl.ds(c * _L, _L)],
                      rankl[pl.ds(c * _L, _L)])
        slotv[pl.ds(c * _L, _L)] = s
        src = lax.iota(jnp.int32, _L) + c * _L
        plsc.store_scatter(idxv, [s], src)
        return 0

    lax.fori_loop(0, nch, p2, 0)

    def p3(c, _):
        ic = idxv[pl.ds(c * _L, _L)]
        posgv[pl.ds(c * _L, _L)] = plsc.load_gather(posv, [ic])
        return 0

    lax.fori_loop(0, tp // _L, p3, 0)

    nlb = nl_pad // BT
    for jj in range(2):
        blkv[pl.ds(jj * _L, _L)] = jnp.where(
            lax.iota(jnp.int32, _L) + jj * _L >= nlb, 1, 0)


def _gather_rows(tab_hbm, out_hbm, idx_ref, base, nrows, chunk, bufs, sems):
    nchunks = nrows // chunk
    cps = [None, None]

    def start(ci):
        b = ci % 2
        cp = pltpu.async_copy(
            tab_hbm.at[idx_ref.at[pl.ds(base + ci * chunk, chunk)]],
            bufs[b], sems[b])
        cps[b] = cp

    start(0)
    for ci in range(nchunks):
        cps[ci % 2].wait()
        if ci + 1 < nchunks:
            start(ci + 1)
        pltpu.sync_copy(bufs[ci % 2], out_hbm.at[pl.ds(base + ci * chunk, chunk)])


def _make_route_gather(t, h, tp):
    rows_w = tp // NW
    chunk = 24
    assert rows_w % chunk == 0 and (tp % NW) == 0
    mesh = plsc.VectorSubcoreMesh(core_axis_name="c", subcore_axis_name="s")

    @functools.partial(
        pl.kernel, mesh=mesh,
        compiler_params=pltpu.CompilerParams(needs_layout_passes=False),
        out_type=(
            jax.ShapeDtypeStruct((tp, h // 2), jnp.int32),
            jax.ShapeDtypeStruct((t,), jnp.int32),
            jax.ShapeDtypeStruct((tp,), jnp.int32),
            jax.ShapeDtypeStruct((tp,), jnp.int32),
            jax.ShapeDtypeStruct((NW,), jnp.int32),
        ),
        scratch_types=[
            pltpu.VMEM((t,), jnp.int32),
            pltpu.VMEM((t,), jnp.int32),
            pltpu.VMEM((t,), jnp.int32),
            pltpu.VMEM((t,), jnp.int32),
            pltpu.VMEM((tp,), jnp.int32),
            pltpu.VMEM((t,), jnp.int32),
            pltpu.VMEM((tp,), jnp.int32),
            pltpu.VMEM((NW,), jnp.int32),
            pltpu.VMEM((chunk, h // 2), jnp.int32),
            pltpu.VMEM((chunk, h // 2), jnp.int32),
            pltpu.SemaphoreType.DMA,
            pltpu.SemaphoreType.DMA,
        ],
    )
    def k(mask_hbm, pos_hbm, hid_hbm, hidg_hbm, slot_hbm, idx_hbm,
          posg_hbm, blk_hbm, maskv, posv, rankv, rankl, idxv, slotv,
          posgv, blkv, buf0, buf1, sem0, sem1):
        wid = _wid()
        _route_compute(t, tp, mask_hbm, pos_hbm, maskv, posv, rankv, rankl,
                       idxv, slotv, posgv, blkv)

        @pl.when(wid == 0)
        def _():
            pltpu.sync_copy(slotv, slot_hbm)
            pltpu.sync_copy(idxv, idx_hbm)
            pltpu.sync_copy(posgv, posg_hbm)
            pltpu.sync_copy(blkv, blk_hbm)

        _gather_rows(hid_hbm, hidg_hbm, idxv, wid * rows_w, rows_w, chunk,
                     (buf0, buf1), (sem0, sem1))

    return k


def _make_gather(b, d, chunk, dtype=jnp.float32):
    rows_w = b // NW
    assert rows_w % chunk == 0 and chunk % 8 == 0
    mesh = plsc.VectorSubcoreMesh(core_axis_name="c", subcore_axis_name="s")

    @functools.partial(
        pl.kernel, mesh=mesh,
        compiler_params=pltpu.CompilerParams(needs_layout_passes=False),
        out_type=jax.ShapeDtypeStruct((b, d), dtype),
        scratch_types=[
            pltpu.VMEM((rows_w,), jnp.int32),
            pltpu.VMEM((chunk, d), dtype),
            pltpu.VMEM((chunk, d), dtype),
            pltpu.SemaphoreType.DMA,
            pltpu.SemaphoreType.DMA,
        ],
    )
    def k(idx_hbm, tab_hbm, out_hbm, idxv, buf0, buf1, sem0, sem1):
        wid = _wid()
        base = wid * rows_w
        pltpu.sync_copy(idx_hbm.at[pl.ds(base, rows_w)], idxv)
        nchunks = rows_w // chunk
        cps = [None, None]

        def start(ci):
            bb = ci % 2
            cps[bb] = pltpu.async_copy(
                tab_hbm.at[idxv.at[pl.ds(ci * chunk, chunk)]],
                (buf0, buf1)[bb], (sem0, sem1)[bb])

        start(0)
        for ci in range(nchunks):
            cps[ci % 2].wait()
            if ci + 1 < nchunks:
                start(ci + 1)
            pltpu.sync_copy((buf0, buf1)[ci % 2],
                            out_hbm.at[pl.ds(base + ci * chunk, chunk)])

    return k


def _rope_block(x, pos_f):
    half = HD // 2
    k = lax.broadcasted_iota(jnp.int32, (1, half), 1).astype(jnp.float32)
    inv_freq = jnp.exp(-(np.log(ROPE_BASE) / half) * k)
    ang = pos_f * inv_freq
    cos = jnp.cos(ang)
    sin = jnp.sin(ang)
    nh = x.shape[1] // HD
    coscat = jnp.concatenate([cos, cos] * nh, axis=1)
    sincat = jnp.concatenate([-sin, sin] * nh, axis=1)
    swapped = jnp.concatenate(
        sum(([x[:, c + half:c + HD], x[:, c:c + half]]
             for c in range(0, x.shape[1], HD)), []), axis=1)
    return x * coscat + swapped * sincat


def _qkv_routed_body(e_ref, h_ref, wv_ref, wl_ref, bv_ref, pos_ref, out_ref):
    j = pl.program_id(0)
    i = pl.program_id(1)
    bt = out_ref.shape[0]
    bn = out_ref.shape[1]
    e = e_ref[i]
    rows = h_ref[pl.ds(i * bt, bt), :]
    acc = lax.cond(
        e == 1,
        lambda: jnp.dot(rows, wv_ref[...],
                        preferred_element_type=jnp.float32),
        lambda: jnp.dot(rows, wl_ref[...],
                        preferred_element_type=jnp.float32))
    acc = acc + e.astype(jnp.float32) * bv_ref[...]
    pos_f = pos_ref[pl.ds(i * bt, bt), :].astype(jnp.float32)
    roped = _rope_block(acc, pos_f)
    out_ref[...] = jnp.where(j < (N_HEADS + N_KV) * HD // bn, roped,
                             acc).astype(jnp.bfloat16)


def _attn_body(q_ref, k_ref, v_ref, o_ref, *, bq, bkv):
    qi = pl.program_id(1)
    scale = 1.0 / np.sqrt(HD)
    q = (q_ref[...].astype(jnp.float32) * scale).astype(jnp.bfloat16)

    def chunk(jj, carry, masked):
        acc, m, l = carry
        kj = k_ref[pl.ds(jj * bkv, bkv), :]
        vj = v_ref[pl.ds(jj * bkv, bkv), :]
        s = lax.dot_general(q, kj, (((1,), (1,)), ((), ())),
                            preferred_element_type=jnp.float32)
        if masked:
            row_l = lax.broadcasted_iota(jnp.int32, (bq, bkv), 0)
            col_l = lax.broadcasted_iota(jnp.int32, (bq, bkv), 1)
            s = jnp.where(col_l <= row_l, s, -1e30)
        m_new = jnp.maximum(m, jnp.max(s, axis=1, keepdims=True))
        p = jnp.exp(s - m_new)
        alpha = jnp.exp(m - m_new)
        l_new = l * alpha + jnp.sum(p, axis=1, keepdims=True)
        acc_new = acc * alpha + jnp.dot(p.astype(jnp.bfloat16), vj,
                                        preferred_element_type=jnp.float32)
        return acc_new, m_new, l_new

    acc0 = jnp.zeros((bq, HD), jnp.float32)
    m0 = jnp.full((bq, 1), -1e30, jnp.float32)
    l0 = jnp.zeros((bq, 1), jnp.float32)
    carry = lax.fori_loop(0, qi, lambda jj, c: chunk(jj, c, False),
                          (acc0, m0, l0))
    acc, m, l = chunk(qi, carry, True)
    o_ref[...] = (acc / l).astype(jnp.bfloat16)


def _dense_routed_body(e_ref, c_ref, wv_ref, wl_ref, out_ref):
    i = pl.program_id(1)
    bt = out_ref.shape[0]
    e = e_ref[i]
    rows = c_ref[pl.ds(i * bt, bt), :]
    out_ref[...] = lax.cond(
        e == 1,
        lambda: jnp.dot(rows, wv_ref[...],
                        preferred_element_type=jnp.float32),
        lambda: jnp.dot(rows, wl_ref[...],
                        preferred_element_type=jnp.float32))


def kernel(hidden_states, positions, vision_token_mask, Wv_qkv, bv_qkv,
           Wl_qkv, Wv_dense, Wl_dense):
    t, h = hidden_states.shape
    qkv = Wv_qkv.shape[1]
    d_out = Wv_dense.shape[1]
    tp = t + 2 * BT
    nb = tp // BT
    mask_i = vision_token_mask.astype(jnp.int32)
    pos_i = positions.astype(jnp.int32)
    bv2d = bv_qkv.reshape(1, qkv)

    def _pack(x):
        n, d = x.shape
        return lax.bitcast_convert_type(x.reshape(n, d // 2, 2), jnp.int32)

    def _unpack(x):
        n, d2 = x.shape
        return lax.bitcast_convert_type(x, jnp.bfloat16).reshape(n, 2 * d2)

    h_pk = _pack(hidden_states.astype(jnp.bfloat16))
    hidden_g, slot, idx, pos_g, blk_exp = _make_route_gather(t, h, tp)(
        mask_i, pos_i, h_pk)
    posg2d = pos_g.reshape(tp, 1)
    hg_bf = _unpack(hidden_g)
    wv_bf = Wv_qkv.astype(jnp.bfloat16)
    wl_bf = Wl_qkv.astype(jnp.bfloat16)

    nj = qkv // BN
    mixed_g = pl.pallas_call(
        _qkv_routed_body,
        grid_spec=pltpu.PrefetchScalarGridSpec(
            num_scalar_prefetch=1,
            grid=(nj, nb),
            in_specs=[
                pl.BlockSpec((tp, h), lambda j, i, e: (0, 0)),
                pl.BlockSpec((h, BN), lambda j, i, e: (0, j)),
                pl.BlockSpec((h, BN), lambda j, i, e: (0, j)),
                pl.BlockSpec((1, BN), lambda j, i, e: (0, j)),
                pl.BlockSpec((tp, 1), lambda j, i, e: (0, 0)),
            ],
            out_specs=pl.BlockSpec((BT, BN), lambda j, i, e: (i, j)),
        ),
        out_shape=jax.ShapeDtypeStruct((tp, qkv), jnp.bfloat16),
        compiler_params=pltpu.CompilerParams(
            dimension_semantics=("arbitrary", "arbitrary")),
    )(blk_exp, hg_bf, wv_bf, wl_bf, bv2d, posg2d)

    mixed = _unpack(_make_gather(t, qkv // 2, 16, jnp.int32)(
        slot, _pack(mixed_g)))

    nq = t // BQ
    ctx = pl.pallas_call(
        functools.partial(_attn_body, bq=BQ, bkv=BKV),
        grid=(N_HEADS, nq),
        in_specs=[
            pl.BlockSpec((BQ, HD), lambda hh, qi: (qi, hh)),
            pl.BlockSpec((t, HD), lambda hh, qi: (0, N_HEADS + hh // 2)),
            pl.BlockSpec((t, HD), lambda hh, qi: (0, N_HEADS + N_KV + hh // 2)),
        ],
        out_specs=pl.BlockSpec((BQ, HD), lambda hh, qi: (qi, hh)),
        out_shape=jax.ShapeDtypeStruct((t, N_HEADS * HD), jnp.bfloat16),
        compiler_params=pltpu.CompilerParams(
            dimension_semantics=("arbitrary", "arbitrary")),
    )(mixed, mixed, mixed)

    ctx_g = _unpack(_make_gather(tp, N_HEADS * HD // 2, 24, jnp.int32)(
        idx, _pack(ctx)))
    wvd_bf = Wv_dense.astype(jnp.bfloat16)
    wld_bf = Wl_dense.astype(jnp.bfloat16)

    nj2 = d_out // BN
    out_g = pl.pallas_call(
        _dense_routed_body,
        grid_spec=pltpu.PrefetchScalarGridSpec(
            num_scalar_prefetch=1,
            grid=(nj2, nb),
            in_specs=[
                pl.BlockSpec((tp, N_HEADS * HD), lambda j, i, e: (0, 0)),
                pl.BlockSpec((N_HEADS * HD, BN), lambda j, i, e: (0, j)),
                pl.BlockSpec((N_HEADS * HD, BN), lambda j, i, e: (0, j)),
            ],
            out_specs=pl.BlockSpec((BT, BN), lambda j, i, e: (i, j)),
        ),
        out_shape=jax.ShapeDtypeStruct((tp, d_out), jnp.float32),
        compiler_params=pltpu.CompilerParams(
            dimension_semantics=("arbitrary", "arbitrary")),
    )(blk_exp, ctx_g, wvd_bf, wld_bf)

    out = _make_gather(t, d_out, 16)(slot, out_g)
    return out


# speedup vs baseline: 3.5840x; 3.5840x over previous
import functools

import jax
import jax.numpy as jnp
import numpy as np
from jax import lax
from jax.experimental import pallas as pl
from jax.experimental.pallas import tpu as pltpu

N_HEADS = 16
N_KV = 8
HD = 128
ROPE_BASE = 500000.0

BT = 256
BN = 256
BQ = 256
BKV = 256


def _rope_block(x, pos_f):
    half = HD // 2
    k = lax.broadcasted_iota(jnp.int32, (1, half), 1).astype(jnp.float32)
    inv_freq = jnp.exp(-(np.log(ROPE_BASE) / half) * k)
    ang = pos_f * inv_freq
    cos = jnp.cos(ang)
    sin = jnp.sin(ang)
    nh = x.shape[1] // HD
    coscat = jnp.concatenate([cos, cos] * nh, axis=1)
    sincat = jnp.concatenate([-sin, sin] * nh, axis=1)
    swapped = jnp.concatenate(
        sum(([x[:, c + half:c + HD], x[:, c:c + half]]
             for c in range(0, x.shape[1], HD)), []), axis=1)
    return x * coscat + swapped * sincat


def _qkv_body(h_ref, wv_ref, wl_ref, bv_ref, mask_ref, pos_ref, out_ref):
    j = pl.program_id(0)
    i = pl.program_id(1)
    bt = out_ref.shape[0]
    rows = h_ref[pl.ds(i * bt, bt), :]
    mv = jnp.dot(rows, wv_ref[...], preferred_element_type=jnp.float32)
    mv = mv + bv_ref[...]
    ml = jnp.dot(rows, wl_ref[...], preferred_element_type=jnp.float32)
    mask = mask_ref[pl.ds(i * bt, bt), :] > 0
    mixed = jnp.where(mask, mv, ml)
    pos_f = pos_ref[pl.ds(i * bt, bt), :].astype(jnp.float32)
    roped = _rope_block(mixed, pos_f)
    out_ref[...] = jnp.where(j < (N_HEADS + N_KV) * HD // out_ref.shape[1],
                             roped, mixed).astype(jnp.bfloat16)


def _attn_body(q_ref, k_ref, v_ref, o_ref, *, bq, bkv):
    qi = pl.program_id(1)
    scale = 1.0 / np.sqrt(HD)
    q2 = (q_ref[...].astype(jnp.float32) * scale).astype(jnp.bfloat16)
    q = jnp.concatenate([q2[:, :HD], q2[:, HD:]], axis=0)

    def chunk(jj, carry, masked):
        acc, m, l = carry
        kj = k_ref[pl.ds(jj * bkv, bkv), :]
        vj = v_ref[pl.ds(jj * bkv, bkv), :]
        s = lax.dot_general(q, kj, (((1,), (1,)), ((), ())),
                            preferred_element_type=jnp.float32)
        if masked:
            row_l = lax.broadcasted_iota(jnp.int32, (2 * bq, bkv), 0) & (bq - 1)
            col_l = lax.broadcasted_iota(jnp.int32, (2 * bq, bkv), 1)
            s = jnp.where(col_l <= row_l, s, -1e30)
        m_new = jnp.maximum(m, jnp.max(s, axis=1, keepdims=True))
        p = jnp.exp(s - m_new)
        alpha = jnp.exp(m - m_new)
        l_new = l * alpha + jnp.sum(p, axis=1, keepdims=True)
        acc_new = acc * alpha + jnp.dot(p.astype(jnp.bfloat16), vj,
                                        preferred_element_type=jnp.float32)
        return acc_new, m_new, l_new

    acc0 = jnp.zeros((2 * bq, HD), jnp.float32)
    m0 = jnp.full((2 * bq, 1), -1e30, jnp.float32)
    l0 = jnp.zeros((2 * bq, 1), jnp.float32)
    carry = lax.fori_loop(0, qi, lambda jj, c: chunk(jj, c, False),
                          (acc0, m0, l0))
    acc, m, l = chunk(qi, carry, True)
    o = acc / l
    o_ref[...] = jnp.concatenate([o[:bq], o[bq:]], axis=1).astype(jnp.bfloat16)


def _dense_body(c_ref, wv_ref, wl_ref, mask_ref, out_ref):
    i = pl.program_id(1)
    bt = out_ref.shape[0]
    rows = c_ref[pl.ds(i * bt, bt), :]
    ov = jnp.dot(rows, wv_ref[...], preferred_element_type=jnp.float32)
    ol = jnp.dot(rows, wl_ref[...], preferred_element_type=jnp.float32)
    mask = mask_ref[pl.ds(i * bt, bt), :] > 0
    out_ref[...] = jnp.where(mask, ov, ol)


def kernel(hidden_states, positions, vision_token_mask, Wv_qkv, bv_qkv,
           Wl_qkv, Wv_dense, Wl_dense):
    t, h = hidden_states.shape
    qkv = Wv_qkv.shape[1]
    d_out = Wv_dense.shape[1]
    mask2d = vision_token_mask.astype(jnp.int32).reshape(t, 1)
    pos2d = positions.astype(jnp.int32).reshape(t, 1)
    bv2d = bv_qkv.reshape(1, qkv)
    h_bf = hidden_states.astype(jnp.bfloat16)
    wv_bf = Wv_qkv.astype(jnp.bfloat16)
    wl_bf = Wl_qkv.astype(jnp.bfloat16)
    wvd_bf = Wv_dense.astype(jnp.bfloat16)
    wld_bf = Wl_dense.astype(jnp.bfloat16)

    nj = qkv // BN
    ni = t // BT
    mixed = pl.pallas_call(
        _qkv_body,
        grid=(nj, ni),
        in_specs=[
            pl.BlockSpec((t, h), lambda j, i: (0, 0)),
            pl.BlockSpec((h, BN), lambda j, i: (0, j)),
            pl.BlockSpec((h, BN), lambda j, i: (0, j)),
            pl.BlockSpec((1, BN), lambda j, i: (0, j)),
            pl.BlockSpec((t, 1), lambda j, i: (0, 0)),
            pl.BlockSpec((t, 1), lambda j, i: (0, 0)),
        ],
        out_specs=pl.BlockSpec((BT, BN), lambda j, i: (i, j)),
        out_shape=jax.ShapeDtypeStruct((t, qkv), jnp.bfloat16),
        compiler_params=pltpu.CompilerParams(
            dimension_semantics=("arbitrary", "arbitrary"),
            allow_input_fusion=(True, True, True, False, False, False)),
    )(h_bf, wv_bf, wl_bf, bv2d, mask2d, pos2d)

    nq = t // BQ
    ctx = pl.pallas_call(
        functools.partial(_attn_body, bq=BQ, bkv=BKV),
        grid=(N_KV, nq),
        in_specs=[
            pl.BlockSpec((BQ, 2 * HD), lambda g, qi: (qi, g)),
            pl.BlockSpec((t, HD), lambda g, qi: (0, N_HEADS + g)),
            pl.BlockSpec((t, HD), lambda g, qi: (0, N_HEADS + N_KV + g)),
        ],
        out_specs=pl.BlockSpec((BQ, 2 * HD), lambda g, qi: (qi, g)),
        out_shape=jax.ShapeDtypeStruct((t, N_HEADS * HD), jnp.bfloat16),
        compiler_params=pltpu.CompilerParams(
            dimension_semantics=("arbitrary", "arbitrary")),
    )(mixed, mixed, mixed)

    nj2 = d_out // BN
    out = pl.pallas_call(
        _dense_body,
        grid=(nj2, ni),
        in_specs=[
            pl.BlockSpec((t, N_HEADS * HD), lambda j, i: (0, 0)),
            pl.BlockSpec((N_HEADS * HD, BN), lambda j, i: (0, j)),
            pl.BlockSpec((N_HEADS * HD, BN), lambda j, i: (0, j)),
            pl.BlockSpec((t, 1), lambda j, i: (0, 0)),
        ],
        out_specs=pl.BlockSpec((BT, BN), lambda j, i: (i, j)),
        out_shape=jax.ShapeDtypeStruct((t, d_out), jnp.float32),
        compiler_params=pltpu.CompilerParams(
            dimension_semantics=("arbitrary", "arbitrary"),
            allow_input_fusion=(False, True, True, False)),
    )(ctx, wvd_bf, wld_bf, mask2d)
    return out


# speedup vs baseline: 4.0687x; 1.1352x over previous
import functools

import jax
import jax.numpy as jnp
import numpy as np
from jax import lax
from jax.experimental import pallas as pl
from jax.experimental.pallas import tpu as pltpu

N_HEADS = 16
N_KV = 8
HD = 128
ROPE_BASE = 500000.0

BT = 256
BN = 256
BQ = 256
BKV = 512


def _rope_block(x, pos_f):
    half = HD // 2
    k = lax.broadcasted_iota(jnp.int32, (1, half), 1).astype(jnp.float32)
    inv_freq = jnp.exp(-(np.log(ROPE_BASE) / half) * k)
    ang = pos_f * inv_freq
    cos = jnp.cos(ang)
    sin = jnp.sin(ang)
    nh = x.shape[1] // HD
    coscat = jnp.concatenate([cos, cos] * nh, axis=1)
    sincat = jnp.concatenate([-sin, sin] * nh, axis=1)
    swapped = jnp.concatenate(
        sum(([x[:, c + half:c + HD], x[:, c:c + half]]
             for c in range(0, x.shape[1], HD)), []), axis=1)
    return x * coscat + swapped * sincat


def _qkv_body(h_ref, wv_ref, wl_ref, bv_ref, mask_ref, pos_ref, out_ref):
    j = pl.program_id(0)
    i = pl.program_id(1)
    bt = out_ref.shape[0]
    rows = h_ref[pl.ds(i * bt, bt), :]
    mv = jnp.dot(rows, wv_ref[...], preferred_element_type=jnp.float32)
    mv = mv + bv_ref[...]
    ml = jnp.dot(rows, wl_ref[...], preferred_element_type=jnp.float32)
    mask = mask_ref[pl.ds(i * bt, bt), :] > 0
    mixed = jnp.where(mask, mv, ml)
    pos_f = pos_ref[pl.ds(i * bt, bt), :].astype(jnp.float32)
    roped = _rope_block(mixed, pos_f)
    out_ref[...] = jnp.where(j < (N_HEADS + N_KV) * HD // out_ref.shape[1],
                             roped, mixed).astype(jnp.bfloat16)


def _attn_body(q_ref, k_ref, v_ref, o_ref, *, bq, bkv):
    qi = pl.program_id(1)
    scale = 1.0 / np.sqrt(HD)
    q2 = (q_ref[...].astype(jnp.float32) * scale).astype(jnp.bfloat16)
    q = jnp.concatenate([q2[:, :HD], q2[:, HD:]], axis=0)

    def chunk(jj, carry, masked):
        acc, m, l = carry
        kj = k_ref[pl.ds(jj * bkv, bkv), :]
        vj = v_ref[pl.ds(jj * bkv, bkv), :]
        s = lax.dot_general(q, kj, (((1,), (1,)), ((), ())),
                            preferred_element_type=jnp.float32)
        if masked:
            row_g = qi * bq + (
                lax.broadcasted_iota(jnp.int32, (2 * bq, bkv), 0) & (bq - 1))
            col_g = jj * bkv + lax.broadcasted_iota(jnp.int32, (2 * bq, bkv), 1)
            s = jnp.where(col_g <= row_g, s, -1e30)
        m_new = jnp.maximum(m, jnp.max(s, axis=1, keepdims=True))
        p = jnp.exp(s - m_new)
        alpha = jnp.exp(m - m_new)
        l_new = l * alpha + jnp.sum(p, axis=1, keepdims=True)
        acc_new = acc * alpha + jnp.dot(p.astype(jnp.bfloat16), vj,
                                        preferred_element_type=jnp.float32)
        return acc_new, m_new, l_new

    acc0 = jnp.zeros((2 * bq, HD), jnp.float32)
    m0 = jnp.full((2 * bq, 1), -1e30, jnp.float32)
    l0 = jnp.zeros((2 * bq, 1), jnp.float32)
    nfull = (qi * bq) // bkv
    carry = lax.fori_loop(0, nfull, lambda jj, c: chunk(jj, c, False),
                          (acc0, m0, l0))
    acc, m, l = chunk(nfull, carry, True)
    o = acc / l
    o_ref[...] = jnp.concatenate([o[:bq], o[bq:]], axis=1).astype(jnp.bfloat16)


def _dense_body(c_ref, wv_ref, wl_ref, mask_ref, out_ref):
    i = pl.program_id(1)
    bt = out_ref.shape[0]
    rows = c_ref[pl.ds(i * bt, bt), :]
    ov = jnp.dot(rows, wv_ref[...], preferred_element_type=jnp.float32)
    ol = jnp.dot(rows, wl_ref[...], preferred_element_type=jnp.float32)
    mask = mask_ref[pl.ds(i * bt, bt), :] > 0
    out_ref[...] = jnp.where(mask, ov, ol)


def kernel(hidden_states, positions, vision_token_mask, Wv_qkv, bv_qkv,
           Wl_qkv, Wv_dense, Wl_dense):
    t, h = hidden_states.shape
    qkv = Wv_qkv.shape[1]
    d_out = Wv_dense.shape[1]
    mask2d = vision_token_mask.astype(jnp.int32).reshape(t, 1)
    pos2d = positions.astype(jnp.int32).reshape(t, 1)
    bv2d = bv_qkv.reshape(1, qkv)
    h_bf = hidden_states.astype(jnp.bfloat16)
    wv_bf = Wv_qkv.astype(jnp.bfloat16)
    wl_bf = Wl_qkv.astype(jnp.bfloat16)
    wvd_bf = Wv_dense.astype(jnp.bfloat16)
    wld_bf = Wl_dense.astype(jnp.bfloat16)

    nj = qkv // BN
    ni = t // BT
    mixed = pl.pallas_call(
        _qkv_body,
        grid=(nj, ni),
        in_specs=[
            pl.BlockSpec((t, h), lambda j, i: (0, 0)),
            pl.BlockSpec((h, BN), lambda j, i: (0, j)),
            pl.BlockSpec((h, BN), lambda j, i: (0, j)),
            pl.BlockSpec((1, BN), lambda j, i: (0, j)),
            pl.BlockSpec((t, 1), lambda j, i: (0, 0)),
            pl.BlockSpec((t, 1), lambda j, i: (0, 0)),
        ],
        out_specs=pl.BlockSpec((BT, BN), lambda j, i: (i, j)),
        out_shape=jax.ShapeDtypeStruct((t, qkv), jnp.bfloat16),
        compiler_params=pltpu.CompilerParams(
            dimension_semantics=("arbitrary", "arbitrary"),
            allow_input_fusion=(True, True, True, False, False, False)),
    )(h_bf, wv_bf, wl_bf, bv2d, mask2d, pos2d)

    nq = t // BQ
    ctx = pl.pallas_call(
        functools.partial(_attn_body, bq=BQ, bkv=BKV),
        grid=(N_KV, nq),
        in_specs=[
            pl.BlockSpec((BQ, 2 * HD), lambda g, qi: (qi, g)),
            pl.BlockSpec((t, HD), lambda g, qi: (0, N_HEADS + g)),
            pl.BlockSpec((t, HD), lambda g, qi: (0, N_HEADS + N_KV + g)),
        ],
        out_specs=pl.BlockSpec((BQ, 2 * HD), lambda g, qi: (qi, g)),
        out_shape=jax.ShapeDtypeStruct((t, N_HEADS * HD), jnp.bfloat16),
        compiler_params=pltpu.CompilerParams(
            dimension_semantics=("arbitrary", "arbitrary")),
    )(mixed, mixed, mixed)

    nj2 = d_out // BN
    out = pl.pallas_call(
        _dense_body,
        grid=(nj2, ni),
        in_specs=[
            pl.BlockSpec((t, N_HEADS * HD), lambda j, i: (0, 0)),
            pl.BlockSpec((N_HEADS * HD, BN), lambda j, i: (0, j)),
            pl.BlockSpec((N_HEADS * HD, BN), lambda j, i: (0, j)),
            pl.BlockSpec((t, 1), lambda j, i: (0, 0)),
        ],
        out_specs=pl.BlockSpec((BT, BN), lambda j, i: (i, j)),
        out_shape=jax.ShapeDtypeStruct((t, d_out), jnp.float32),
        compiler_params=pltpu.CompilerParams(
            dimension_semantics=("arbitrary", "arbitrary"),
            allow_input_fusion=(False, True, True, False)),
    )(ctx, wvd_bf, wld_bf, mask2d)
    return out


# speedup vs baseline: 4.5814x; 1.1260x over previous
import functools

import jax
import jax.numpy as jnp
import numpy as np
from jax import lax
from jax.experimental import pallas as pl
from jax.experimental.pallas import tpu as pltpu

N_HEADS = 16
N_KV = 8
HD = 128
ROPE_BASE = 500000.0

BT = 512
BN = 512
BQ = 256
BKV = 512


def _rope_block(x, pos_f):
    half = HD // 2
    k = lax.broadcasted_iota(jnp.int32, (1, half), 1).astype(jnp.float32)
    inv_freq = jnp.exp(-(np.log(ROPE_BASE) / half) * k)
    ang = pos_f * inv_freq
    cos = jnp.cos(ang)
    sin = jnp.sin(ang)
    nh = x.shape[1] // HD
    coscat = jnp.concatenate([cos, cos] * nh, axis=1)
    sincat = jnp.concatenate([-sin, sin] * nh, axis=1)
    swapped = jnp.concatenate(
        sum(([x[:, c + half:c + HD], x[:, c:c + half]]
             for c in range(0, x.shape[1], HD)), []), axis=1)
    return x * coscat + swapped * sincat


def _qkv_body(h_ref, wv_ref, wl_ref, bv_ref, mask_ref, pos_ref, out_ref):
    j = pl.program_id(0)
    i = pl.program_id(1)
    bt = out_ref.shape[0]
    rows = h_ref[pl.ds(i * bt, bt), :]
    mv = jnp.dot(rows, wv_ref[...], preferred_element_type=jnp.float32)
    mv = mv + bv_ref[...]
    ml = jnp.dot(rows, wl_ref[...], preferred_element_type=jnp.float32)
    mask = mask_ref[pl.ds(i * bt, bt), :] > 0
    mixed = jnp.where(mask, mv, ml)
    pos_f = pos_ref[pl.ds(i * bt, bt), :].astype(jnp.float32)
    roped = _rope_block(mixed, pos_f)
    out_ref[...] = jnp.where(j < (N_HEADS + N_KV) * HD // out_ref.shape[1],
                             roped, mixed).astype(jnp.bfloat16)


def _attn_body(q_ref, k_ref, v_ref, o_ref, *, bq, bkv):
    qi = pl.program_id(1)
    scale = 1.0 / np.sqrt(HD)
    q2 = (q_ref[...].astype(jnp.float32) * scale).astype(jnp.bfloat16)
    q = jnp.concatenate([q2[:, :HD], q2[:, HD:]], axis=0)

    def chunk(jj, carry, masked):
        acc, m, l = carry
        kj = k_ref[pl.ds(jj * bkv, bkv), :]
        vj = v_ref[pl.ds(jj * bkv, bkv), :]
        s = lax.dot_general(q, kj, (((1,), (1,)), ((), ())),
                            preferred_element_type=jnp.float32)
        if masked:
            row_g = qi * bq + (
                lax.broadcasted_iota(jnp.int32, (2 * bq, bkv), 0) & (bq - 1))
            col_g = jj * bkv + lax.broadcasted_iota(jnp.int32, (2 * bq, bkv), 1)
            s = jnp.where(col_g <= row_g, s, -1e30)
        m_new = jnp.maximum(m, jnp.max(s, axis=1, keepdims=True))
        p = jnp.exp(s - m_new)
        alpha = jnp.exp(m - m_new)
        l_new = l * alpha + jnp.sum(p, axis=1, keepdims=True)
        acc_new = acc * alpha + jnp.dot(p.astype(jnp.bfloat16), vj,
                                        preferred_element_type=jnp.float32)
        return acc_new, m_new, l_new

    acc0 = jnp.zeros((2 * bq, HD), jnp.float32)
    m0 = jnp.full((2 * bq, 1), -1e30, jnp.float32)
    l0 = jnp.zeros((2 * bq, 1), jnp.float32)
    nfull = (qi * bq) // bkv
    carry = lax.fori_loop(0, nfull, lambda jj, c: chunk(jj, c, False),
                          (acc0, m0, l0))
    acc, m, l = chunk(nfull, carry, True)
    o = acc / l
    o_ref[...] = jnp.concatenate([o[:bq], o[bq:]], axis=1).astype(jnp.bfloat16)


def _dense_body(c_ref, wv_ref, wl_ref, mask_ref, out_ref):
    i = pl.program_id(1)
    bt = out_ref.shape[0]
    rows = c_ref[pl.ds(i * bt, bt), :]
    ov = jnp.dot(rows, wv_ref[...], preferred_element_type=jnp.float32)
    ol = jnp.dot(rows, wl_ref[...], preferred_element_type=jnp.float32)
    mask = mask_ref[pl.ds(i * bt, bt), :] > 0
    out_ref[...] = jnp.where(mask, ov, ol)


def kernel(hidden_states, positions, vision_token_mask, Wv_qkv, bv_qkv,
           Wl_qkv, Wv_dense, Wl_dense):
    t, h = hidden_states.shape
    qkv = Wv_qkv.shape[1]
    d_out = Wv_dense.shape[1]
    mask2d = vision_token_mask.astype(jnp.int32).reshape(t, 1)
    pos2d = positions.astype(jnp.int32).reshape(t, 1)
    bv2d = bv_qkv.reshape(1, qkv)
    h_bf = hidden_states.astype(jnp.bfloat16)
    wv_bf = Wv_qkv.astype(jnp.bfloat16)
    wl_bf = Wl_qkv.astype(jnp.bfloat16)
    wvd_bf = Wv_dense.astype(jnp.bfloat16)
    wld_bf = Wl_dense.astype(jnp.bfloat16)

    nj = qkv // BN
    ni = t // BT
    mixed = pl.pallas_call(
        _qkv_body,
        grid=(nj, ni),
        in_specs=[
            pl.BlockSpec((t, h), lambda j, i: (0, 0)),
            pl.BlockSpec((h, BN), lambda j, i: (0, j)),
            pl.BlockSpec((h, BN), lambda j, i: (0, j)),
            pl.BlockSpec((1, BN), lambda j, i: (0, j)),
            pl.BlockSpec((t, 1), lambda j, i: (0, 0)),
            pl.BlockSpec((t, 1), lambda j, i: (0, 0)),
        ],
        out_specs=pl.BlockSpec((BT, BN), lambda j, i: (i, j)),
        out_shape=jax.ShapeDtypeStruct((t, qkv), jnp.bfloat16),
        compiler_params=pltpu.CompilerParams(
            dimension_semantics=("arbitrary", "arbitrary"),
            allow_input_fusion=(True, True, True, False, False, False)),
    )(h_bf, wv_bf, wl_bf, bv2d, mask2d, pos2d)

    nq = t // BQ
    ctx = pl.pallas_call(
        functools.partial(_attn_body, bq=BQ, bkv=BKV),
        grid=(N_KV, nq),
        in_specs=[
            pl.BlockSpec((BQ, 2 * HD), lambda g, qi: (qi, g)),
            pl.BlockSpec((t, HD), lambda g, qi: (0, N_HEADS + g)),
            pl.BlockSpec((t, HD), lambda g, qi: (0, N_HEADS + N_KV + g)),
        ],
        out_specs=pl.BlockSpec((BQ, 2 * HD), lambda g, qi: (qi, g)),
        out_shape=jax.ShapeDtypeStruct((t, N_HEADS * HD), jnp.bfloat16),
        compiler_params=pltpu.CompilerParams(
            dimension_semantics=("arbitrary", "arbitrary")),
    )(mixed, mixed, mixed)

    nj2 = d_out // BN
    out = pl.pallas_call(
        _dense_body,
        grid=(nj2, ni),
        in_specs=[
            pl.BlockSpec((t, N_HEADS * HD), lambda j, i: (0, 0)),
            pl.BlockSpec((N_HEADS * HD, BN), lambda j, i: (0, j)),
            pl.BlockSpec((N_HEADS * HD, BN), lambda j, i: (0, j)),
            pl.BlockSpec((t, 1), lambda j, i: (0, 0)),
        ],
        out_specs=pl.BlockSpec((BT, BN), lambda j, i: (i, j)),
        out_shape=jax.ShapeDtypeStruct((t, d_out), jnp.float32),
        compiler_params=pltpu.CompilerParams(
            dimension_semantics=("arbitrary", "arbitrary"),
            allow_input_fusion=(False, True, True, False)),
    )(ctx, wvd_bf, wld_bf, mask2d)
    return out


# speedup vs baseline: 4.8221x; 1.0525x over previous
import functools

import jax
import jax.numpy as jnp
import numpy as np
from jax import lax
from jax.experimental import pallas as pl
from jax.experimental.pallas import tpu as pltpu

N_HEADS = 16
N_KV = 8
HD = 128
ROPE_BASE = 500000.0

BT = 512
BN = 1024
BQ = 256
BKV = 512


def _rope_block(x, pos_f):
    half = HD // 2
    k = lax.broadcasted_iota(jnp.int32, (1, half), 1).astype(jnp.float32)
    inv_freq = jnp.exp(-(np.log(ROPE_BASE) / half) * k)
    ang = pos_f * inv_freq
    cos = jnp.cos(ang)
    sin = jnp.sin(ang)
    nh = x.shape[1] // HD
    coscat = jnp.concatenate([cos, cos] * nh, axis=1)
    sincat = jnp.concatenate([-sin, sin] * nh, axis=1)
    swapped = jnp.concatenate(
        sum(([x[:, c + half:c + HD], x[:, c:c + half]]
             for c in range(0, x.shape[1], HD)), []), axis=1)
    return x * coscat + swapped * sincat


def _qkv_body(h_ref, wv_ref, wl_ref, bv_ref, mask_ref, pos_ref, out_ref):
    j = pl.program_id(0)
    i = pl.program_id(1)
    bt = out_ref.shape[0]
    rows = h_ref[pl.ds(i * bt, bt), :]
    mv = jnp.dot(rows, wv_ref[...], preferred_element_type=jnp.float32)
    mv = mv + bv_ref[...]
    ml = jnp.dot(rows, wl_ref[...], preferred_element_type=jnp.float32)
    mask = mask_ref[pl.ds(i * bt, bt), :] > 0
    mixed = jnp.where(mask, mv, ml)
    pos_f = pos_ref[pl.ds(i * bt, bt), :].astype(jnp.float32)
    roped = _rope_block(mixed, pos_f)
    out_ref[...] = jnp.where(j < (N_HEADS + N_KV) * HD // out_ref.shape[1],
                             roped, mixed).astype(jnp.bfloat16)


def _attn_body(q_ref, k_ref, v_ref, o_ref, *, bq, bkv):
    qi = pl.program_id(1)
    scale = 1.0 / np.sqrt(HD)
    q2 = (q_ref[...].astype(jnp.float32) * scale).astype(jnp.bfloat16)
    q = jnp.concatenate([q2[:, :HD], q2[:, HD:]], axis=0)

    def chunk(jj, carry, masked):
        acc, m, l = carry
        kj = k_ref[pl.ds(jj * bkv, bkv), :]
        vj = v_ref[pl.ds(jj * bkv, bkv), :]
        s = lax.dot_general(q, kj, (((1,), (1,)), ((), ())),
                            preferred_element_type=jnp.float32)
        if masked:
            row_g = qi * bq + (
                lax.broadcasted_iota(jnp.int32, (2 * bq, bkv), 0) & (bq - 1))
            col_g = jj * bkv + lax.broadcasted_iota(jnp.int32, (2 * bq, bkv), 1)
            s = jnp.where(col_g <= row_g, s, -1e30)
        m_new = jnp.maximum(m, jnp.max(s, axis=1, keepdims=True))
        p = jnp.exp(s - m_new)
        alpha = jnp.exp(m - m_new)
        l_new = l * alpha + jnp.sum(p, axis=1, keepdims=True)
        acc_new = acc * alpha + jnp.dot(p.astype(jnp.bfloat16), vj,
                                        preferred_element_type=jnp.float32)
        return acc_new, m_new, l_new

    acc0 = jnp.zeros((2 * bq, HD), jnp.float32)
    m0 = jnp.full((2 * bq, 1), -1e30, jnp.float32)
    l0 = jnp.zeros((2 * bq, 1), jnp.float32)
    nfull = (qi * bq) // bkv
    carry = lax.fori_loop(0, nfull, lambda jj, c: chunk(jj, c, False),
                          (acc0, m0, l0))
    acc, m, l = chunk(nfull, carry, True)
    o = acc / l
    o_ref[...] = jnp.concatenate([o[:bq], o[bq:]], axis=1).astype(jnp.bfloat16)


def _dense_body(c_ref, wv_ref, wl_ref, mask_ref, out_ref):
    i = pl.program_id(1)
    bt = out_ref.shape[0]
    rows = c_ref[pl.ds(i * bt, bt), :]
    ov = jnp.dot(rows, wv_ref[...], preferred_element_type=jnp.float32)
    ol = jnp.dot(rows, wl_ref[...], preferred_element_type=jnp.float32)
    mask = mask_ref[pl.ds(i * bt, bt), :] > 0
    out_ref[...] = jnp.where(mask, ov, ol)


def kernel(hidden_states, positions, vision_token_mask, Wv_qkv, bv_qkv,
           Wl_qkv, Wv_dense, Wl_dense):
    t, h = hidden_states.shape
    qkv = Wv_qkv.shape[1]
    d_out = Wv_dense.shape[1]
    mask2d = vision_token_mask.astype(jnp.int32).reshape(t, 1)
    pos2d = positions.astype(jnp.int32).reshape(t, 1)
    bv2d = bv_qkv.reshape(1, qkv)
    h_bf = hidden_states.astype(jnp.bfloat16)
    wv_bf = Wv_qkv.astype(jnp.bfloat16)
    wl_bf = Wl_qkv.astype(jnp.bfloat16)
    wvd_bf = Wv_dense.astype(jnp.bfloat16)
    wld_bf = Wl_dense.astype(jnp.bfloat16)

    nj = qkv // BN
    ni = t // BT
    mixed = pl.pallas_call(
        _qkv_body,
        grid=(nj, ni),
        in_specs=[
            pl.BlockSpec((t, h), lambda j, i: (0, 0)),
            pl.BlockSpec((h, BN), lambda j, i: (0, j)),
            pl.BlockSpec((h, BN), lambda j, i: (0, j)),
            pl.BlockSpec((1, BN), lambda j, i: (0, j)),
            pl.BlockSpec((t, 1), lambda j, i: (0, 0)),
            pl.BlockSpec((t, 1), lambda j, i: (0, 0)),
        ],
        out_specs=pl.BlockSpec((BT, BN), lambda j, i: (i, j)),
        out_shape=jax.ShapeDtypeStruct((t, qkv), jnp.bfloat16),
        compiler_params=pltpu.CompilerParams(
            dimension_semantics=("arbitrary", "arbitrary"),
            allow_input_fusion=(True, True, True, False, False, False)),
    )(h_bf, wv_bf, wl_bf, bv2d, mask2d, pos2d)

    nq = t // BQ
    ctx = pl.pallas_call(
        functools.partial(_attn_body, bq=BQ, bkv=BKV),
        grid=(N_KV, nq),
        in_specs=[
            pl.BlockSpec((BQ, 2 * HD), lambda g, qi: (qi, g)),
            pl.BlockSpec((t, HD), lambda g, qi: (0, N_HEADS + g)),
            pl.BlockSpec((t, HD), lambda g, qi: (0, N_HEADS + N_KV + g)),
        ],
        out_specs=pl.BlockSpec((BQ, 2 * HD), lambda g, qi: (qi, g)),
        out_shape=jax.ShapeDtypeStruct((t, N_HEADS * HD), jnp.bfloat16),
        compiler_params=pltpu.CompilerParams(
            dimension_semantics=("arbitrary", "arbitrary")),
    )(mixed, mixed, mixed)

    nj2 = d_out // BN
    out = pl.pallas_call(
        _dense_body,
        grid=(nj2, ni),
        in_specs=[
            pl.BlockSpec((t, N_HEADS * HD), lambda j, i: (0, 0)),
            pl.BlockSpec((N_HEADS * HD, BN), lambda j, i: (0, j)),
            pl.BlockSpec((N_HEADS * HD, BN), lambda j, i: (0, j)),
            pl.BlockSpec((t, 1), lambda j, i: (0, 0)),
        ],
        out_specs=pl.BlockSpec((BT, BN), lambda j, i: (i, j)),
        out_shape=jax.ShapeDtypeStruct((t, d_out), jnp.float32),
        compiler_params=pltpu.CompilerParams(
            dimension_semantics=("arbitrary", "arbitrary"),
            allow_input_fusion=(False, True, True, False)),
    )(ctx, wvd_bf, wld_bf, mask2d)
    return out


# speedup vs baseline: 5.1151x; 1.0608x over previous
import functools

import jax
import jax.numpy as jnp
import numpy as np
from jax import lax
from jax.experimental import pallas as pl
from jax.experimental.pallas import tpu as pltpu

N_HEADS = 16
N_KV = 8
HD = 128
ROPE_BASE = 500000.0

BT = 512
BN = 1024
BQ = 512
BKV = 512


def _rope_block(x, pos_f):
    half = HD // 2
    k = lax.broadcasted_iota(jnp.int32, (1, half), 1).astype(jnp.float32)
    inv_freq = jnp.exp(-(np.log(ROPE_BASE) / half) * k)
    ang = pos_f * inv_freq
    cos = jnp.cos(ang)
    sin = jnp.sin(ang)
    nh = x.shape[1] // HD
    coscat = jnp.concatenate([cos, cos] * nh, axis=1)
    sincat = jnp.concatenate([-sin, sin] * nh, axis=1)
    swapped = jnp.concatenate(
        sum(([x[:, c + half:c + HD], x[:, c:c + half]]
             for c in range(0, x.shape[1], HD)), []), axis=1)
    return x * coscat + swapped * sincat


def _qkv_body(h_ref, wv_ref, wl_ref, bv_ref, mask_ref, pos_ref, out_ref):
    j = pl.program_id(0)
    i = pl.program_id(1)
    bt = out_ref.shape[0]
    rows = h_ref[pl.ds(i * bt, bt), :]
    mv = jnp.dot(rows, wv_ref[...], preferred_element_type=jnp.float32)
    mv = mv + bv_ref[...]
    ml = jnp.dot(rows, wl_ref[...], preferred_element_type=jnp.float32)
    mask = mask_ref[pl.ds(i * bt, bt), :] > 0
    mixed = jnp.where(mask, mv, ml)
    pos_f = pos_ref[pl.ds(i * bt, bt), :].astype(jnp.float32)
    roped = _rope_block(mixed, pos_f)
    out_ref[...] = jnp.where(j < (N_HEADS + N_KV) * HD // out_ref.shape[1],
                             roped, mixed).astype(jnp.bfloat16)


def _attn_body(q_ref, k_ref, v_ref, o_ref, *, bq, bkv):
    qi = pl.program_id(1)
    scale = 1.0 / np.sqrt(HD)
    q2 = (q_ref[...].astype(jnp.float32) * scale).astype(jnp.bfloat16)
    q = jnp.concatenate([q2[:, :HD], q2[:, HD:]], axis=0)

    def chunk(jj, carry, masked):
        acc, m, l = carry
        kj = k_ref[pl.ds(jj * bkv, bkv), :]
        vj = v_ref[pl.ds(jj * bkv, bkv), :]
        s = lax.dot_general(q, kj, (((1,), (1,)), ((), ())),
                            preferred_element_type=jnp.float32)
        if masked:
            row_g = qi * bq + (
                lax.broadcasted_iota(jnp.int32, (2 * bq, bkv), 0) & (bq - 1))
            col_g = jj * bkv + lax.broadcasted_iota(jnp.int32, (2 * bq, bkv), 1)
            s = jnp.where(col_g <= row_g, s, -1e30)
        m_new = jnp.maximum(m, jnp.max(s, axis=1, keepdims=True))
        p = jnp.exp(s - m_new)
        alpha = jnp.exp(m - m_new)
        l_new = l * alpha + jnp.sum(p, axis=1, keepdims=True)
        acc_new = acc * alpha + jnp.dot(p.astype(jnp.bfloat16), vj,
                                        preferred_element_type=jnp.float32)
        return acc_new, m_new, l_new

    acc0 = jnp.zeros((2 * bq, HD), jnp.float32)
    m0 = jnp.full((2 * bq, 1), -1e30, jnp.float32)
    l0 = jnp.zeros((2 * bq, 1), jnp.float32)
    nfull = (qi * bq) // bkv
    carry = lax.fori_loop(0, nfull, lambda jj, c: chunk(jj, c, False),
                          (acc0, m0, l0))
    acc, m, l = chunk(nfull, carry, True)
    o = acc / l
    o_ref[...] = jnp.concatenate([o[:bq], o[bq:]], axis=1).astype(jnp.bfloat16)


def _dense_body(c_ref, wv_ref, wl_ref, mask_ref, out_ref):
    i = pl.program_id(1)
    bt = out_ref.shape[0]
    rows = c_ref[pl.ds(i * bt, bt), :]
    ov = jnp.dot(rows, wv_ref[...], preferred_element_type=jnp.float32)
    ol = jnp.dot(rows, wl_ref[...], preferred_element_type=jnp.float32)
    mask = mask_ref[pl.ds(i * bt, bt), :] > 0
    out_ref[...] = jnp.where(mask, ov, ol)


def kernel(hidden_states, positions, vision_token_mask, Wv_qkv, bv_qkv,
           Wl_qkv, Wv_dense, Wl_dense):
    t, h = hidden_states.shape
    qkv = Wv_qkv.shape[1]
    d_out = Wv_dense.shape[1]
    mask2d = vision_token_mask.astype(jnp.int32).reshape(t, 1)
    pos2d = positions.astype(jnp.int32).reshape(t, 1)
    bv2d = bv_qkv.reshape(1, qkv)
    h_bf = hidden_states.astype(jnp.bfloat16)
    wv_bf = Wv_qkv.astype(jnp.bfloat16)
    wl_bf = Wl_qkv.astype(jnp.bfloat16)
    wvd_bf = Wv_dense.astype(jnp.bfloat16)
    wld_bf = Wl_dense.astype(jnp.bfloat16)

    nj = qkv // BN
    ni = t // BT
    mixed = pl.pallas_call(
        _qkv_body,
        grid=(nj, ni),
        in_specs=[
            pl.BlockSpec((t, h), lambda j, i: (0, 0)),
            pl.BlockSpec((h, BN), lambda j, i: (0, j)),
            pl.BlockSpec((h, BN), lambda j, i: (0, j)),
            pl.BlockSpec((1, BN), lambda j, i: (0, j)),
            pl.BlockSpec((t, 1), lambda j, i: (0, 0)),
            pl.BlockSpec((t, 1), lambda j, i: (0, 0)),
        ],
        out_specs=pl.BlockSpec((BT, BN), lambda j, i: (i, j)),
        out_shape=jax.ShapeDtypeStruct((t, qkv), jnp.bfloat16),
        compiler_params=pltpu.CompilerParams(
            dimension_semantics=("arbitrary", "arbitrary"),
            allow_input_fusion=(True, True, True, False, False, False)),
    )(h_bf, wv_bf, wl_bf, bv2d, mask2d, pos2d)

    nq = t // BQ
    ctx = pl.pallas_call(
        functools.partial(_attn_body, bq=BQ, bkv=BKV),
        grid=(N_KV, nq),
        in_specs=[
            pl.BlockSpec((BQ, 2 * HD), lambda g, qi: (qi, g)),
            pl.BlockSpec((t, HD), lambda g, qi: (0, N_HEADS + g)),
            pl.BlockSpec((t, HD), lambda g, qi: (0, N_HEADS + N_KV + g)),
        ],
        out_specs=pl.BlockSpec((BQ, 2 * HD), lambda g, qi: (qi, g)),
        out_shape=jax.ShapeDtypeStruct((t, N_HEADS * HD), jnp.bfloat16),
        compiler_params=pltpu.CompilerParams(
            dimension_semantics=("arbitrary", "arbitrary")),
    )(mixed, mixed, mixed)

    nj2 = d_out // BN
    out = pl.pallas_call(
        _dense_body,
        grid=(nj2, ni),
        in_specs=[
            pl.BlockSpec((t, N_HEADS * HD), lambda j, i: (0, 0)),
            pl.BlockSpec((N_HEADS * HD, BN), lambda j, i: (0, j)),
            pl.BlockSpec((N_HEADS * HD, BN), lambda j, i: (0, j)),
            pl.BlockSpec((t, 1), lambda j, i: (0, 0)),
        ],
        out_specs=pl.BlockSpec((BT, BN), lambda j, i: (i, j)),
        out_shape=jax.ShapeDtypeStruct((t, d_out), jnp.float32),
        compiler_params=pltpu.CompilerParams(
            dimension_semantics=("arbitrary", "arbitrary"),
            allow_input_fusion=(False, True, True, False)),
    )(ctx, wvd_bf, wld_bf, mask2d)
    return out
